# Initial kernel scaffold; baseline (speedup 1.0000x reference)
import jax
import jax.numpy as jnp
from jax.experimental import pallas as pl


def kernel(action, predict, label):
    raise NotImplementedError("write your pallas kernel here")


# speedup vs baseline: 29.7977x; 29.7977x over previous
import functools

import jax
import jax.numpy as jnp
from jax.experimental import pallas as pl
from jax.experimental.pallas import tpu as pltpu

_LOG1P25 = 0.22314355131420976


def _tc_body(pred_ref, lab_ref, act_ref, out_ref):
    i = pl.program_id(0)
    x = act_ref[...]
    m = jnp.max(x, axis=1, keepdims=True)
    s = jnp.sum(jnp.exp(x - m), axis=1)
    val = 1.0 / s
    t = jnp.log1p(-0.2 * val) + _LOG1P25
    part = jnp.sum(t * t)

    @pl.when(i == 0)
    def _init():
        out_ref[0, 0] = 0.0

    out_ref[0, 0] += part

    @pl.when(i == pl.num_programs(0) - 1)
    def _fin():
        p0 = pred_ref[0, 0]
        p1 = pred_ref[0, 1]
        gate_off = (p1 > p0) & (lab_ref[0] == 1)
        n_rows = pl.num_programs(0) * x.shape[0]
        out_ref[0, 0] = jnp.where(gate_off, 0.0, out_ref[0, 0] / n_rows)


@jax.jit
def kernel(action, predict, label):
    n, d = action.shape
    block = 256
    out = pl.pallas_call(
        _tc_body,
        grid=(n // block,),
        in_specs=[
            pl.BlockSpec(memory_space=pltpu.SMEM),
            pl.BlockSpec(memory_space=pltpu.SMEM),
            pl.BlockSpec((block, d), lambda i: (i, 0)),
        ],
        out_specs=pl.BlockSpec(memory_space=pltpu.SMEM),
        out_shape=jax.ShapeDtypeStruct((1, 1), jnp.float32),
    )(predict, label, action)
    return out[0, 0]
